# baseline (speedup 1.0000x reference)
import functools

import jax
import jax.numpy as jnp
from jax import lax
from jax.experimental import pallas as pl
from jax.experimental.pallas import tpu as pltpu
from jax.experimental.pallas import tpu_sc as plsc

_N = 10000
_E = 320000
_D = 128
_DEPTH = 4

_NC = 2
_NS = 16
_NW = _NC * _NS
_CHUNK = 128
_NCHUNKS = 80
_EW = _CHUNK * _NCHUNKS
_EP = _EW * _NW
_ACC_N = 10240
_RPS = _ACC_N // _NS
_TRASH = 10016

_mesh = plsc.VectorSubcoreMesh(core_axis_name="c", subcore_axis_name="s")


def _sc_agg(hs, srcp, dstp, zeros):
  @functools.partial(
      pl.kernel,
      out_type=jax.ShapeDtypeStruct((_NC * _ACC_N, _D), jnp.float32),
      mesh=_mesh,
      scratch_types=[
          pltpu.VMEM_SHARED((_ACC_N, _D), jnp.float32),
          pltpu.VMEM((_CHUNK,), jnp.int32),
          pltpu.VMEM((_CHUNK,), jnp.int32),
          pltpu.VMEM((_CHUNK,), jnp.int32),
          pltpu.VMEM((_CHUNK,), jnp.int32),
          pltpu.VMEM((_CHUNK, _D), jnp.float32),
          pltpu.VMEM((_CHUNK, _D), jnp.float32),
          pltpu.SemaphoreType.DMA,
          pltpu.SemaphoreType.DMA,
          pltpu.SemaphoreType.DMA,
          pltpu.SemaphoreType.DMA,
      ],
  )
  def k(hs_hbm, src_hbm, dst_hbm, zeros_hbm, out_hbm, acc, src0, src1,
        dst0, dst1, rows0, rows1, gsem0, gsem1, isem0, isem1):
    c = lax.axis_index("c")
    s = lax.axis_index("s")
    base = (c * _NS + s) * _EW
    row0 = s * _RPS
    pltpu.sync_copy(zeros_hbm.at[pl.ds(row0, _RPS)], acc.at[pl.ds(row0, _RPS)])

    srcb = (src0, src1)
    dstb = (dst0, dst1)
    rowsb = (rows0, rows1)
    gsems = (gsem0, gsem1)
    isems = (isem0, isem1)

    pltpu.sync_copy(src_hbm.at[pl.ds(base, _CHUNK)], src0)
    pltpu.sync_copy(dst_hbm.at[pl.ds(base, _CHUNK)], dst0)
    pltpu.async_copy(src_hbm.at[pl.ds(base + _CHUNK, _CHUNK)], src1, isem1)
    pltpu.async_copy(dst_hbm.at[pl.ds(base + _CHUNK, _CHUNK)], dst1, isem1)
    plsc.subcore_barrier()
    pltpu.async_copy(hs_hbm.at[src0], rows0, gsem0)

    @pl.loop(0, _NCHUNKS // 2)
    def _(i):
      for b in range(2):
        t = 2 * i + b
        o = 1 - b
        pltpu.make_async_copy(hs_hbm.at[srcb[b]], rowsb[b], gsems[b]).wait()

        @pl.when(t + 1 < _NCHUNKS)
        def _():
          pltpu.make_async_copy(src_hbm.at[pl.ds(base, _CHUNK)], srcb[o],
                                isems[o]).wait()
          pltpu.make_async_copy(dst_hbm.at[pl.ds(base, _CHUNK)], dstb[o],
                                isems[o]).wait()
          pltpu.async_copy(hs_hbm.at[srcb[o]], rowsb[o], gsems[o])

        pltpu.sync_copy(rowsb[b], acc.at[dstb[b]], add=True)

        @pl.when(t + 2 < _NCHUNKS)
        def _():
          off = base + (t + 2) * _CHUNK
          pltpu.async_copy(src_hbm.at[pl.ds(off, _CHUNK)], srcb[b], isems[b])
          pltpu.async_copy(dst_hbm.at[pl.ds(off, _CHUNK)], dstb[b], isems[b])

    plsc.subcore_barrier()
    pltpu.sync_copy(acc.at[pl.ds(row0, _RPS)],
                    out_hbm.at[pl.ds(c * _ACC_N + row0, _RPS)])

  return k(hs, srcp, dstp, zeros)


def _sc_deg(dstp, ones, zeros):
  @functools.partial(
      pl.kernel,
      out_type=jax.ShapeDtypeStruct((_NC * _ACC_N, _D), jnp.float32),
      mesh=_mesh,
      scratch_types=[
          pltpu.VMEM_SHARED((_ACC_N, _D), jnp.float32),
          pltpu.VMEM((_CHUNK,), jnp.int32),
          pltpu.VMEM((_CHUNK,), jnp.int32),
          pltpu.VMEM((_CHUNK, _D), jnp.float32),
          pltpu.SemaphoreType.DMA,
          pltpu.SemaphoreType.DMA,
      ],
  )
  def k(dst_hbm, ones_hbm, zeros_hbm, out_hbm, acc, dst0, dst1, ones_v,
        isem0, isem1):
    c = lax.axis_index("c")
    s = lax.axis_index("s")
    base = (c * _NS + s) * _EW
    row0 = s * _RPS
    pltpu.sync_copy(zeros_hbm.at[pl.ds(row0, _RPS)], acc.at[pl.ds(row0, _RPS)])
    pltpu.sync_copy(ones_hbm, ones_v)
    dstb = (dst0, dst1)
    isems = (isem0, isem1)
    pltpu.sync_copy(dst_hbm.at[pl.ds(base, _CHUNK)], dst0)
    pltpu.async_copy(dst_hbm.at[pl.ds(base + _CHUNK, _CHUNK)], dst1, isem1)
    plsc.subcore_barrier()

    @pl.loop(0, _NCHUNKS // 2)
    def _(i):
      for b in range(2):
        t = 2 * i + b

        @pl.when(t > 0)
        def _():
          pltpu.make_async_copy(dst_hbm.at[pl.ds(base, _CHUNK)], dstb[b],
                                isems[b]).wait()

        pltpu.sync_copy(ones_v, acc.at[dstb[b]], add=True)

        @pl.when(t + 2 < _NCHUNKS)
        def _():
          off = base + (t + 2) * _CHUNK
          pltpu.async_copy(dst_hbm.at[pl.ds(off, _CHUNK)], dstb[b], isems[b])

    plsc.subcore_barrier()
    pltpu.sync_copy(acc.at[pl.ds(row0, _RPS)],
                    out_hbm.at[pl.ds(c * _ACC_N + row0, _RPS)])

  return k(dstp, ones, zeros)


def _proj_body(x_ref, w1_ref, b1_ref, w2_ref, b2_ref, o_ref):
  h = jnp.dot(x_ref[...], w1_ref[...], preferred_element_type=jnp.float32)
  h = jnp.maximum(h + b1_ref[...], 0.0)
  y = jnp.dot(h, w2_ref[...], preferred_element_type=jnp.float32)
  o_ref[...] = jax.nn.gelu(y + b2_ref[...])


def _dinv_body(deg_ref, o_ref):
  d0 = deg_ref[0:_N, 0:1]
  d1 = deg_ref[_ACC_N:_ACC_N + _N, 0:1]
  o_ref[...] = lax.rsqrt(d0 + d1 + 1.0)


def _pre_body(x_ref, w_ref, dinv_ref, o_ref):
  h = jnp.dot(x_ref[...], w_ref[...], preferred_element_type=jnp.float32)
  o_ref[...] = h * dinv_ref[...]


def _post_body(parts_ref, hs_ref, dinv_ref, b_ref, g_ref, bb_ref, o_ref,
               *, last):
  agg = parts_ref[0:_N, :] + parts_ref[_ACC_N:_ACC_N + _N, :]
  y = dinv_ref[...] * (agg + hs_ref[...]) + b_ref[...]
  if last:
    o_ref[...] = y
  else:
    y = jax.nn.gelu(y)
    mu = jnp.mean(y, axis=-1, keepdims=True)
    yc = y - mu
    var = jnp.mean(yc * yc, axis=-1, keepdims=True)
    o_ref[...] = yc * lax.rsqrt(var + 1e-5) * g_ref[...] + bb_ref[...]


def _dec_body(x_ref, w1_ref, b1_ref, w2_ref, b2_ref, o_ref):
  h = jnp.dot(x_ref[...], w1_ref[...], preferred_element_type=jnp.float32)
  h = jnp.maximum(h + b1_ref[...], 0.0)
  y = jnp.dot(h, w2_ref[...], preferred_element_type=jnp.float32)
  o_ref[...] = y + b2_ref[...]


def _tc(body, n_out_cols, *args):
  return pl.pallas_call(
      body,
      out_shape=jax.ShapeDtypeStruct((_N, n_out_cols), jnp.float32),
  )(*args)


def kernel(nodes, grid, edge_index, edge_attr, batch_size, image_size,
           proj_w1, proj_b1, proj_w2, proj_b2, gcn_w, gcn_b, ln_g, ln_b,
           dec_w1, dec_b1, dec_w2, dec_b2):
  del edge_attr, batch_size, image_size

  x0 = jnp.concatenate([nodes, grid], axis=-1)
  x0 = jnp.pad(x0, ((0, 0), (0, 16 - x0.shape[1])))
  w1p = jnp.pad(proj_w1, ((0, 16 - proj_w1.shape[0]), (0, 0)))

  src = edge_index[0]
  dst = edge_index[1]
  npad = _EP - _E
  srcp = jnp.concatenate([src, jnp.zeros((npad,), jnp.int32)])
  dstp = jnp.concatenate([dst, jnp.full((npad,), _TRASH, jnp.int32)])

  zeros = jnp.zeros((_ACC_N, _D), jnp.float32)
  ones = jnp.ones((_CHUNK, _D), jnp.float32)

  b1 = proj_b1.reshape(1, -1)
  b2 = proj_b2.reshape(1, -1)
  g = ln_g.reshape(1, -1)
  bb = ln_b.reshape(1, -1)
  db1 = dec_b1.reshape(1, -1)
  db2 = dec_b2.reshape(1, -1)

  deg_parts = _sc_deg(dstp, ones, zeros)
  x = _tc(_proj_body, _D, x0, w1p, b1, proj_w2, b2)
  dinv = _tc(_dinv_body, 1, deg_parts)

  for i in range(_DEPTH):
    hs = _tc(_pre_body, _D, x, gcn_w[i], dinv)
    parts = _sc_agg(hs, srcp, dstp, zeros)
    x = _tc(
        functools.partial(_post_body, last=(i == _DEPTH - 1)),
        _D, parts, hs, dinv, gcn_b[i].reshape(1, -1), g, bb)

  return _tc(_dec_body, 1, x, dec_w1, db1, dec_w2, db2)


# speedup vs baseline: 1.3906x; 1.3906x over previous
import functools

import jax
import jax.numpy as jnp
from jax import lax
from jax.experimental import pallas as pl
from jax.experimental.pallas import tpu as pltpu
from jax.experimental.pallas import tpu_sc as plsc

_N = 10000
_E = 320000
_D = 128
_DEPTH = 4

_NC = 2
_NS = 16
_NW = _NC * _NS
_CHUNK = 128
_NCHUNKS = 80
_EW = _CHUNK * _NCHUNKS
_EP = _EW * _NW
_ACC_N = 10240
_RPS = _ACC_N // _NS
_TRASH = 10016

_mesh = plsc.VectorSubcoreMesh(core_axis_name="c", subcore_axis_name="s")


def _sc_agg(hs, sd, zeros):
  @functools.partial(
      pl.kernel,
      out_type=jax.ShapeDtypeStruct((_NC * _ACC_N, _D), jnp.float32),
      mesh=_mesh,
      scratch_types=[
          pltpu.VMEM_SHARED((_ACC_N, _D), jnp.float32),
          pltpu.VMEM((2, _CHUNK), jnp.int32),
          pltpu.VMEM((2, _CHUNK), jnp.int32),
          pltpu.VMEM((2, _CHUNK), jnp.int32),
          pltpu.VMEM((_CHUNK, _D), jnp.float32),
          pltpu.VMEM((_CHUNK, _D), jnp.float32),
          pltpu.SemaphoreType.DMA,
          pltpu.SemaphoreType.DMA,
          pltpu.SemaphoreType.DMA,
          pltpu.SemaphoreType.DMA,
          pltpu.SemaphoreType.DMA,
          pltpu.SemaphoreType.DMA,
          pltpu.SemaphoreType.DMA,
      ],
  )
  def k(hs_hbm, sd_hbm, zeros_hbm, out_hbm, acc, i0, i1, i2, rows0, rows1,
        gsem0, gsem1, ssem0, ssem1, isem0, isem1, isem2):
    c = lax.axis_index("c")
    s = lax.axis_index("s")
    wrow = (c * _NS + s) * _NCHUNKS
    row0 = s * _RPS
    pltpu.sync_copy(zeros_hbm.at[pl.ds(row0, _RPS)], acc.at[pl.ds(row0, _RPS)])

    ib = (i0, i1, i2)
    rowsb = (rows0, rows1)
    gsems = (gsem0, gsem1)
    ssems = (ssem0, ssem1)
    isems = (isem0, isem1, isem2)

    def idx_load(t):
      return pltpu.async_copy(sd_hbm.at[pl.ds(2 * (wrow + t), 2)],
                              ib[t % 3], isems[t % 3])

    pltpu.sync_copy(sd_hbm.at[pl.ds(2 * wrow, 2)], i0)
    iD = {1: idx_load(1)}
    plsc.subcore_barrier()
    gD = {0: pltpu.async_copy(hs_hbm.at[i0.at[0]], rows0, gsem0)}
    sD = {}

    for t in range(_NCHUNKS):
      b = t % 2
      if t >= 1:
        sD[t - 1].wait()
      if t + 2 < _NCHUNKS:
        iD[t + 2] = idx_load(t + 2)
      if t + 1 < _NCHUNKS:
        iD[t + 1].wait()
        gD[t + 1] = pltpu.async_copy(hs_hbm.at[ib[(t + 1) % 3].at[0]],
                                     rowsb[1 - b], gsems[1 - b])
      gD[t].wait()
      sD[t] = pltpu.async_copy(rowsb[b], acc.at[ib[t % 3].at[1]], ssems[b],
                               add=True)
    sD[_NCHUNKS - 1].wait()

    plsc.subcore_barrier()
    pltpu.sync_copy(acc.at[pl.ds(row0, _RPS)],
                    out_hbm.at[pl.ds(c * _ACC_N + row0, _RPS)])

  return k(hs, sd, zeros)


def _sc_deg(dstp, ones, zeros):
  @functools.partial(
      pl.kernel,
      out_type=jax.ShapeDtypeStruct((_NC * _ACC_N, _D), jnp.float32),
      mesh=_mesh,
      scratch_types=[
          pltpu.VMEM_SHARED((_ACC_N, _D), jnp.float32),
          pltpu.VMEM((_CHUNK,), jnp.int32),
          pltpu.VMEM((_CHUNK,), jnp.int32),
          pltpu.VMEM((_CHUNK, _D), jnp.float32),
          pltpu.SemaphoreType.DMA,
          pltpu.SemaphoreType.DMA,
      ],
  )
  def k(dst_hbm, ones_hbm, zeros_hbm, out_hbm, acc, dst0, dst1, ones_v,
        isem0, isem1):
    c = lax.axis_index("c")
    s = lax.axis_index("s")
    base = (c * _NS + s) * _EW
    row0 = s * _RPS
    pltpu.sync_copy(zeros_hbm.at[pl.ds(row0, _RPS)], acc.at[pl.ds(row0, _RPS)])
    pltpu.sync_copy(ones_hbm, ones_v)
    dstb = (dst0, dst1)
    isems = (isem0, isem1)
    pltpu.sync_copy(dst_hbm.at[pl.ds(base, _CHUNK)], dst0)
    pltpu.async_copy(dst_hbm.at[pl.ds(base + _CHUNK, _CHUNK)], dst1, isem1)
    plsc.subcore_barrier()

    @pl.loop(0, _NCHUNKS // 2)
    def _(i):
      for b in range(2):
        t = 2 * i + b

        @pl.when(t > 0)
        def _():
          pltpu.make_async_copy(dst_hbm.at[pl.ds(base, _CHUNK)], dstb[b],
                                isems[b]).wait()

        pltpu.sync_copy(ones_v, acc.at[dstb[b]], add=True)

        @pl.when(t + 2 < _NCHUNKS)
        def _():
          off = base + (t + 2) * _CHUNK
          pltpu.async_copy(dst_hbm.at[pl.ds(off, _CHUNK)], dstb[b], isems[b])

    plsc.subcore_barrier()
    pltpu.sync_copy(acc.at[pl.ds(row0, _RPS)],
                    out_hbm.at[pl.ds(c * _ACC_N + row0, _RPS)])

  return k(dstp, ones, zeros)


def _proj_body(x_ref, w1_ref, b1_ref, w2_ref, b2_ref, o_ref):
  h = jnp.dot(x_ref[...], w1_ref[...], preferred_element_type=jnp.float32)
  h = jnp.maximum(h + b1_ref[...], 0.0)
  y = jnp.dot(h, w2_ref[...], preferred_element_type=jnp.float32)
  o_ref[...] = jax.nn.gelu(y + b2_ref[...])


def _dinv_body(deg_ref, o_ref):
  d0 = deg_ref[0:_N, 0:1]
  d1 = deg_ref[_ACC_N:_ACC_N + _N, 0:1]
  o_ref[...] = lax.rsqrt(d0 + d1 + 1.0)


def _pre_body(x_ref, w_ref, dinv_ref, o_ref):
  h = jnp.dot(x_ref[...], w_ref[...], preferred_element_type=jnp.float32)
  o_ref[...] = h * dinv_ref[...]


def _post_body(parts_ref, hs_ref, dinv_ref, b_ref, g_ref, bb_ref, o_ref,
               *, last):
  agg = parts_ref[0:_N, :] + parts_ref[_ACC_N:_ACC_N + _N, :]
  y = dinv_ref[...] * (agg + hs_ref[...]) + b_ref[...]
  if last:
    o_ref[...] = y
  else:
    y = jax.nn.gelu(y)
    mu = jnp.mean(y, axis=-1, keepdims=True)
    yc = y - mu
    var = jnp.mean(yc * yc, axis=-1, keepdims=True)
    o_ref[...] = yc * lax.rsqrt(var + 1e-5) * g_ref[...] + bb_ref[...]


def _dec_body(x_ref, w1_ref, b1_ref, w2_ref, b2_ref, o_ref):
  h = jnp.dot(x_ref[...], w1_ref[...], preferred_element_type=jnp.float32)
  h = jnp.maximum(h + b1_ref[...], 0.0)
  y = jnp.dot(h, w2_ref[...], preferred_element_type=jnp.float32)
  o_ref[...] = y + b2_ref[...]


def _tc(body, n_out_cols, *args):
  return pl.pallas_call(
      body,
      out_shape=jax.ShapeDtypeStruct((_N, n_out_cols), jnp.float32),
  )(*args)


def kernel(nodes, grid, edge_index, edge_attr, batch_size, image_size,
           proj_w1, proj_b1, proj_w2, proj_b2, gcn_w, gcn_b, ln_g, ln_b,
           dec_w1, dec_b1, dec_w2, dec_b2):
  del edge_attr, batch_size, image_size

  x0 = jnp.concatenate([nodes, grid], axis=-1)
  x0 = jnp.pad(x0, ((0, 0), (0, 16 - x0.shape[1])))
  w1p = jnp.pad(proj_w1, ((0, 16 - proj_w1.shape[0]), (0, 0)))

  src = edge_index[0]
  dst = edge_index[1]
  npad = _EP - _E
  srcp = jnp.concatenate([src, jnp.zeros((npad,), jnp.int32)])
  dstp = jnp.concatenate([dst, jnp.full((npad,), _TRASH, jnp.int32)])
  sd = jnp.stack([srcp.reshape(-1, _CHUNK), dstp.reshape(-1, _CHUNK)],
                 axis=1).reshape(-1, _CHUNK)

  zeros = jnp.zeros((_ACC_N, _D), jnp.float32)
  ones = jnp.ones((_CHUNK, _D), jnp.float32)

  b1 = proj_b1.reshape(1, -1)
  b2 = proj_b2.reshape(1, -1)
  g = ln_g.reshape(1, -1)
  bb = ln_b.reshape(1, -1)
  db1 = dec_b1.reshape(1, -1)
  db2 = dec_b2.reshape(1, -1)

  deg_parts = _sc_deg(dstp, ones, zeros)
  x = _tc(_proj_body, _D, x0, w1p, b1, proj_w2, b2)
  dinv = _tc(_dinv_body, 1, deg_parts)

  for i in range(_DEPTH):
    hs = _tc(_pre_body, _D, x, gcn_w[i], dinv)
    parts = _sc_agg(hs, sd, zeros)
    x = _tc(
        functools.partial(_post_body, last=(i == _DEPTH - 1)),
        _D, parts, hs, dinv, gcn_b[i].reshape(1, -1), g, bb)

  return _tc(_dec_body, 1, x, dec_w1, db1, dec_w2, db2)


# speedup vs baseline: 3.9151x; 2.8154x over previous
import functools

import jax
import jax.numpy as jnp
from jax import lax
from jax.experimental import pallas as pl
from jax.experimental.pallas import tpu as pltpu
from jax.experimental.pallas import tpu_sc as plsc

_N = 10000
_E = 320000
_D = 128
_DEPTH = 4

_NC = 2
_NS = 16
_NW = _NC * _NS
_CHUNK = 128
_NCHUNKS = 79
_EW = _CHUNK * _NCHUNKS
_EP = _EW * _NW
_ACC_N = 10240
_RPS = _ACC_N // _NS
_NTRASH = _ACC_N - _N

_mesh = plsc.VectorSubcoreMesh(core_axis_name="c", subcore_axis_name="s")


def _sc_agg(hs, sd, zeros):
  @functools.partial(
      pl.kernel,
      out_type=jax.ShapeDtypeStruct((_NC * _ACC_N, _D), jnp.float32),
      mesh=_mesh,
      scratch_types=[
          pltpu.VMEM_SHARED((_ACC_N, _D), jnp.float32),
          pltpu.VMEM((2, _CHUNK), jnp.int32),
          pltpu.VMEM((2, _CHUNK), jnp.int32),
          pltpu.VMEM((2, _CHUNK), jnp.int32),
          pltpu.VMEM((_CHUNK, _D), jnp.float32),
          pltpu.VMEM((_CHUNK, _D), jnp.float32),
          pltpu.SemaphoreType.DMA,
          pltpu.SemaphoreType.DMA,
          pltpu.SemaphoreType.DMA,
          pltpu.SemaphoreType.DMA,
          pltpu.SemaphoreType.DMA,
          pltpu.SemaphoreType.DMA,
          pltpu.SemaphoreType.DMA,
      ],
  )
  def k(hs_hbm, sd_hbm, zeros_hbm, out_hbm, acc, i0, i1, i2, rows0, rows1,
        gsem0, gsem1, ssem0, ssem1, isem0, isem1, isem2):
    c = lax.axis_index("c")
    s = lax.axis_index("s")
    wrow = (c * _NS + s) * _NCHUNKS
    row0 = s * _RPS
    pltpu.sync_copy(zeros_hbm.at[pl.ds(row0, _RPS)], acc.at[pl.ds(row0, _RPS)])

    ib = (i0, i1, i2)
    rowsb = (rows0, rows1)
    gsems = (gsem0, gsem1)
    ssems = (ssem0, ssem1)
    isems = (isem0, isem1, isem2)

    def idx_load(t):
      return pltpu.async_copy(sd_hbm.at[pl.ds(2 * (wrow + t), 2)],
                              ib[t % 3], isems[t % 3])

    pltpu.sync_copy(sd_hbm.at[pl.ds(2 * wrow, 2)], i0)
    iD = {1: idx_load(1)}
    plsc.subcore_barrier()
    gD = {0: pltpu.async_copy(hs_hbm.at[i0.at[0]], rows0, gsem0)}
    sD = {}

    for t in range(_NCHUNKS):
      b = t % 2
      if t >= 1:
        sD[t - 1].wait()
      if t + 2 < _NCHUNKS:
        iD[t + 2] = idx_load(t + 2)
      if t + 1 < _NCHUNKS:
        iD[t + 1].wait()
        gD[t + 1] = pltpu.async_copy(hs_hbm.at[ib[(t + 1) % 3].at[0]],
                                     rowsb[1 - b], gsems[1 - b])
      gD[t].wait()
      sD[t] = pltpu.async_copy(rowsb[b], acc.at[ib[t % 3].at[1]], ssems[b],
                               add=True)
    sD[_NCHUNKS - 1].wait()

    plsc.subcore_barrier()
    pltpu.sync_copy(acc.at[pl.ds(row0, _RPS)],
                    out_hbm.at[pl.ds(c * _ACC_N + row0, _RPS)])

  return k(hs, sd, zeros)


def _sc_deg(sd, ones, zeros):
  @functools.partial(
      pl.kernel,
      out_type=jax.ShapeDtypeStruct((_NC * _ACC_N, _D), jnp.float32),
      mesh=_mesh,
      scratch_types=[
          pltpu.VMEM_SHARED((_ACC_N, _D), jnp.float32),
          pltpu.VMEM((1, _CHUNK), jnp.int32),
          pltpu.VMEM((1, _CHUNK), jnp.int32),
          pltpu.VMEM((_CHUNK, _D), jnp.float32),
          pltpu.SemaphoreType.DMA,
          pltpu.SemaphoreType.DMA,
      ],
  )
  def k(sd_hbm, ones_hbm, zeros_hbm, out_hbm, acc, i0, i1, ones_v,
        isem0, isem1):
    c = lax.axis_index("c")
    s = lax.axis_index("s")
    wrow = (c * _NS + s) * _NCHUNKS
    row0 = s * _RPS
    pltpu.sync_copy(zeros_hbm.at[pl.ds(row0, _RPS)], acc.at[pl.ds(row0, _RPS)])
    pltpu.sync_copy(ones_hbm, ones_v)
    ib = (i0, i1)
    isems = (isem0, isem1)

    def idx_load(t):
      return pltpu.async_copy(sd_hbm.at[pl.ds(2 * (wrow + t) + 1, 1)],
                              ib[t % 2], isems[t % 2])

    pltpu.sync_copy(sd_hbm.at[pl.ds(2 * wrow + 1, 1)], i0)
    iD = {1: idx_load(1)}
    plsc.subcore_barrier()

    for t in range(_NCHUNKS):
      if t >= 1:
        iD[t].wait()
      pltpu.sync_copy(ones_v, acc.at[ib[t % 2].at[0]], add=True)
      if t + 2 < _NCHUNKS:
        iD[t + 2] = idx_load(t + 2)

    plsc.subcore_barrier()
    pltpu.sync_copy(acc.at[pl.ds(row0, _RPS)],
                    out_hbm.at[pl.ds(c * _ACC_N + row0, _RPS)])

  return k(sd, ones, zeros)


def _proj_body(x_ref, w1_ref, b1_ref, w2_ref, b2_ref, o_ref):
  h = jnp.dot(x_ref[...], w1_ref[...], preferred_element_type=jnp.float32)
  h = jnp.maximum(h + b1_ref[...], 0.0)
  y = jnp.dot(h, w2_ref[...], preferred_element_type=jnp.float32)
  o_ref[...] = jax.nn.gelu(y + b2_ref[...])


def _dinv_body(deg_ref, o_ref):
  d0 = deg_ref[0:_N, 0:1]
  d1 = deg_ref[_ACC_N:_ACC_N + _N, 0:1]
  o_ref[...] = lax.rsqrt(d0 + d1 + 1.0)


def _pre_body(x_ref, w_ref, dinv_ref, o_ref):
  h = jnp.dot(x_ref[...], w_ref[...], preferred_element_type=jnp.float32)
  o_ref[...] = h * dinv_ref[...]


def _post_body(parts_ref, hs_ref, dinv_ref, b_ref, g_ref, bb_ref, o_ref,
               *, last):
  agg = parts_ref[0:_N, :] + parts_ref[_ACC_N:_ACC_N + _N, :]
  y = dinv_ref[...] * (agg + hs_ref[...]) + b_ref[...]
  if last:
    o_ref[...] = y
  else:
    y = jax.nn.gelu(y)
    mu = jnp.mean(y, axis=-1, keepdims=True)
    yc = y - mu
    var = jnp.mean(yc * yc, axis=-1, keepdims=True)
    o_ref[...] = yc * lax.rsqrt(var + 1e-5) * g_ref[...] + bb_ref[...]


def _dec_body(x_ref, w1_ref, b1_ref, w2_ref, b2_ref, o_ref):
  h = jnp.dot(x_ref[...], w1_ref[...], preferred_element_type=jnp.float32)
  h = jnp.maximum(h + b1_ref[...], 0.0)
  y = jnp.dot(h, w2_ref[...], preferred_element_type=jnp.float32)
  o_ref[...] = y + b2_ref[...]


def _tc(body, n_out_cols, *args):
  return pl.pallas_call(
      body,
      out_shape=jax.ShapeDtypeStruct((_N, n_out_cols), jnp.float32),
  )(*args)


def kernel(nodes, grid, edge_index, edge_attr, batch_size, image_size,
           proj_w1, proj_b1, proj_w2, proj_b2, gcn_w, gcn_b, ln_g, ln_b,
           dec_w1, dec_b1, dec_w2, dec_b2):
  del edge_attr, batch_size, image_size

  x0 = jnp.concatenate([nodes, grid], axis=-1)
  x0 = jnp.pad(x0, ((0, 0), (0, 16 - x0.shape[1])))
  w1p = jnp.pad(proj_w1, ((0, 16 - proj_w1.shape[0]), (0, 0)))

  src = edge_index[0]
  dst = edge_index[1]
  npad = _EP - _E
  pad_ar = jnp.arange(npad, dtype=jnp.int32)
  srcp = jnp.concatenate([src, (pad_ar * 131) % _N])
  dstp = jnp.concatenate([dst, _N + pad_ar % _NTRASH])
  sd = jnp.stack([srcp.reshape(-1, _CHUNK), dstp.reshape(-1, _CHUNK)],
                 axis=1).reshape(-1, _CHUNK)

  zeros = jnp.zeros((_ACC_N, _D), jnp.float32)
  ones = jnp.ones((_CHUNK, _D), jnp.float32)

  b1 = proj_b1.reshape(1, -1)
  b2 = proj_b2.reshape(1, -1)
  g = ln_g.reshape(1, -1)
  bb = ln_b.reshape(1, -1)
  db1 = dec_b1.reshape(1, -1)
  db2 = dec_b2.reshape(1, -1)

  deg_parts = _sc_deg(sd, ones, zeros)
  x = _tc(_proj_body, _D, x0, w1p, b1, proj_w2, b2)
  dinv = _tc(_dinv_body, 1, deg_parts)

  for i in range(_DEPTH):
    hs = _tc(_pre_body, _D, x, gcn_w[i], dinv)
    parts = _sc_agg(hs, sd, zeros)
    x = _tc(
        functools.partial(_post_body, last=(i == _DEPTH - 1)),
        _D, parts, hs, dinv, gcn_b[i].reshape(1, -1), g, bb)

  return _tc(_dec_body, 1, x, dec_w1, db1, dec_w2, db2)


# speedup vs baseline: 4.1353x; 1.0562x over previous
import functools

import jax
import jax.numpy as jnp
from jax import lax
from jax.experimental import pallas as pl
from jax.experimental.pallas import tpu as pltpu
from jax.experimental.pallas import tpu_sc as plsc

_N = 10000
_E = 320000
_D = 128
_DEPTH = 4

_NC = 2
_NS = 16
_NW = _NC * _NS
_CHUNK = 128
_NCHUNKS = 79
_EW = _CHUNK * _NCHUNKS
_EP = _EW * _NW
_ACC_N = 10240
_RPS = _ACC_N // _NS
_NTRASH = _ACC_N - _N

_mesh = plsc.VectorSubcoreMesh(core_axis_name="c", subcore_axis_name="s")


def _sc_agg(hs, sd, zeros):
  @functools.partial(
      pl.kernel,
      out_type=jax.ShapeDtypeStruct((_NC * _ACC_N, _D), jnp.float32),
      mesh=_mesh,
      scratch_types=[
          pltpu.VMEM_SHARED((_ACC_N, _D), jnp.float32),
          pltpu.VMEM((2, _CHUNK), jnp.int32),
          pltpu.VMEM((2, _CHUNK), jnp.int32),
          pltpu.VMEM((2, _CHUNK), jnp.int32),
          pltpu.VMEM((_CHUNK, _D), jnp.float32),
          pltpu.VMEM((_CHUNK, _D), jnp.float32),
          pltpu.SemaphoreType.DMA,
          pltpu.SemaphoreType.DMA,
          pltpu.SemaphoreType.DMA,
          pltpu.SemaphoreType.DMA,
          pltpu.SemaphoreType.DMA,
          pltpu.SemaphoreType.DMA,
          pltpu.SemaphoreType.DMA,
      ],
  )
  def k(hs_hbm, sd_hbm, zeros_hbm, out_hbm, acc, i0, i1, i2, rows0, rows1,
        gsem0, gsem1, ssem0, ssem1, isem0, isem1, isem2):
    c = lax.axis_index("c")
    s = lax.axis_index("s")
    wrow = (c * _NS + s) * _NCHUNKS
    row0 = s * _RPS
    pltpu.sync_copy(zeros_hbm.at[pl.ds(row0, _RPS)], acc.at[pl.ds(row0, _RPS)])

    ib = (i0, i1, i2)
    rowsb = (rows0, rows1)
    gsems = (gsem0, gsem1)
    ssems = (ssem0, ssem1)
    isems = (isem0, isem1, isem2)

    def idx_load(t):
      return pltpu.async_copy(sd_hbm.at[pl.ds(2 * (wrow + t), 2)],
                              ib[t % 3], isems[t % 3])

    pltpu.sync_copy(sd_hbm.at[pl.ds(2 * wrow, 2)], i0)
    iD = {1: idx_load(1)}
    plsc.subcore_barrier()
    gD = {0: pltpu.async_copy(hs_hbm.at[i0.at[0]], rows0, gsem0)}
    sD = {}

    for t in range(_NCHUNKS):
      b = t % 2
      if t >= 1:
        sD[t - 1].wait()
      if t + 2 < _NCHUNKS:
        iD[t + 2] = idx_load(t + 2)
      if t + 1 < _NCHUNKS:
        iD[t + 1].wait()
        gD[t + 1] = pltpu.async_copy(hs_hbm.at[ib[(t + 1) % 3].at[0]],
                                     rowsb[1 - b], gsems[1 - b])
      gD[t].wait()
      sD[t] = pltpu.async_copy(rowsb[b], acc.at[ib[t % 3].at[1]], ssems[b],
                               add=True)
    sD[_NCHUNKS - 1].wait()

    plsc.subcore_barrier()
    pltpu.sync_copy(acc.at[pl.ds(row0, _RPS)],
                    out_hbm.at[pl.ds(c * _ACC_N + row0, _RPS)])

  return k(hs, sd, zeros)


def _sc_deg(sd, ones, zeros):
  @functools.partial(
      pl.kernel,
      out_type=jax.ShapeDtypeStruct((_NC * _ACC_N, _D), jnp.float32),
      mesh=_mesh,
      scratch_types=[
          pltpu.VMEM_SHARED((_ACC_N, _D), jnp.float32),
          pltpu.VMEM((1, _CHUNK), jnp.int32),
          pltpu.VMEM((1, _CHUNK), jnp.int32),
          pltpu.VMEM((1, _CHUNK), jnp.int32),
          pltpu.VMEM((1, _CHUNK), jnp.int32),
          pltpu.VMEM((_CHUNK, _D), jnp.float32),
          pltpu.SemaphoreType.DMA,
          pltpu.SemaphoreType.DMA,
          pltpu.SemaphoreType.DMA,
          pltpu.SemaphoreType.DMA,
          pltpu.SemaphoreType.DMA,
          pltpu.SemaphoreType.DMA,
      ],
  )
  def k(sd_hbm, ones_hbm, zeros_hbm, out_hbm, acc, i0, i1, i2, i3, ones_v,
        isem0, isem1, isem2, isem3, ssem0, ssem1):
    c = lax.axis_index("c")
    s = lax.axis_index("s")
    wrow = (c * _NS + s) * _NCHUNKS
    row0 = s * _RPS
    pltpu.sync_copy(zeros_hbm.at[pl.ds(row0, _RPS)], acc.at[pl.ds(row0, _RPS)])
    pltpu.sync_copy(ones_hbm, ones_v)
    ib = (i0, i1, i2, i3)
    isems = (isem0, isem1, isem2, isem3)
    ssems = (ssem0, ssem1)

    def idx_load(t):
      return pltpu.async_copy(sd_hbm.at[pl.ds(2 * (wrow + t) + 1, 1)],
                              ib[t % 4], isems[t % 4])

    pltpu.sync_copy(sd_hbm.at[pl.ds(2 * wrow + 1, 1)], i0)
    iD = {1: idx_load(1)}
    plsc.subcore_barrier()
    sD = {}

    for t in range(_NCHUNKS):
      if t >= 2:
        sD[t - 2].wait()
      if t + 2 < _NCHUNKS:
        iD[t + 2] = idx_load(t + 2)
      if t >= 1:
        iD[t].wait()
      sD[t] = pltpu.async_copy(ones_v, acc.at[ib[t % 4].at[0]],
                               ssems[t % 2], add=True)
    sD[_NCHUNKS - 2].wait()
    sD[_NCHUNKS - 1].wait()

    plsc.subcore_barrier()
    pltpu.sync_copy(acc.at[pl.ds(row0, _RPS)],
                    out_hbm.at[pl.ds(c * _ACC_N + row0, _RPS)])

  return k(sd, ones, zeros)


def _proj_body(x_ref, w1_ref, b1_ref, w2_ref, b2_ref, o_ref):
  h = jnp.dot(x_ref[...], w1_ref[...], preferred_element_type=jnp.float32)
  h = jnp.maximum(h + b1_ref[...], 0.0)
  y = jnp.dot(h, w2_ref[...], preferred_element_type=jnp.float32)
  o_ref[...] = jax.nn.gelu(y + b2_ref[...])


def _dinvpre_body(deg_ref, x_ref, w_ref, hs_ref, dinv_ref):
  d0 = deg_ref[0:_N, 0:1]
  d1 = deg_ref[_ACC_N:_ACC_N + _N, 0:1]
  dinv = lax.rsqrt(d0 + d1 + 1.0)
  dinv_ref[...] = dinv
  h = jnp.dot(x_ref[...], w_ref[...], preferred_element_type=jnp.float32)
  hs_ref[...] = h * dinv


def _postpre_body(parts_ref, hs_ref, dinv_ref, b_ref, g_ref, bb_ref, w_ref,
                  o_ref):
  agg = parts_ref[0:_N, :] + parts_ref[_ACC_N:_ACC_N + _N, :]
  dinv = dinv_ref[...]
  y = dinv * (agg + hs_ref[...]) + b_ref[...]
  y = jax.nn.gelu(y)
  mu = jnp.mean(y, axis=-1, keepdims=True)
  yc = y - mu
  var = jnp.mean(yc * yc, axis=-1, keepdims=True)
  x = yc * lax.rsqrt(var + 1e-5) * g_ref[...] + bb_ref[...]
  h = jnp.dot(x, w_ref[...], preferred_element_type=jnp.float32)
  o_ref[...] = h * dinv


def _postdec_body(parts_ref, hs_ref, dinv_ref, b_ref, w1_ref, b1_ref,
                  w2_ref, b2_ref, o_ref):
  agg = parts_ref[0:_N, :] + parts_ref[_ACC_N:_ACC_N + _N, :]
  y = dinv_ref[...] * (agg + hs_ref[...]) + b_ref[...]
  h = jnp.dot(y, w1_ref[...], preferred_element_type=jnp.float32)
  h = jnp.maximum(h + b1_ref[...], 0.0)
  z = jnp.dot(h, w2_ref[...], preferred_element_type=jnp.float32)
  o_ref[...] = z + b2_ref[...]


def _tc(body, out_shape, *args):
  return pl.pallas_call(body, out_shape=out_shape)(*args)


def _sds(cols):
  return jax.ShapeDtypeStruct((_N, cols), jnp.float32)


def kernel(nodes, grid, edge_index, edge_attr, batch_size, image_size,
           proj_w1, proj_b1, proj_w2, proj_b2, gcn_w, gcn_b, ln_g, ln_b,
           dec_w1, dec_b1, dec_w2, dec_b2):
  del edge_attr, batch_size, image_size

  x0 = jnp.concatenate([nodes, grid], axis=-1)
  x0 = jnp.pad(x0, ((0, 0), (0, 16 - x0.shape[1])))
  w1p = jnp.pad(proj_w1, ((0, 16 - proj_w1.shape[0]), (0, 0)))

  src = edge_index[0]
  dst = edge_index[1]
  npad = _EP - _E
  pad_ar = jnp.arange(npad, dtype=jnp.int32)
  srcp = jnp.concatenate([src, (pad_ar * 131) % _N])
  dstp = jnp.concatenate([dst, _N + pad_ar % _NTRASH])
  sd = jnp.stack([srcp.reshape(-1, _CHUNK), dstp.reshape(-1, _CHUNK)],
                 axis=1).reshape(-1, _CHUNK)

  zeros = jnp.zeros((_ACC_N, _D), jnp.float32)
  ones = jnp.ones((_CHUNK, _D), jnp.float32)

  b1 = proj_b1.reshape(1, -1)
  b2 = proj_b2.reshape(1, -1)
  g = ln_g.reshape(1, -1)
  bb = ln_b.reshape(1, -1)
  db1 = dec_b1.reshape(1, -1)
  db2 = dec_b2.reshape(1, -1)

  deg_parts = _sc_deg(sd, ones, zeros)
  x = _tc(_proj_body, _sds(_D), x0, w1p, b1, proj_w2, b2)
  hs, dinv = _tc(_dinvpre_body, (_sds(_D), _sds(1)), deg_parts, x, gcn_w[0])

  for i in range(_DEPTH - 1):
    parts = _sc_agg(hs, sd, zeros)
    hs = _tc(_postpre_body, _sds(_D), parts, hs, dinv,
             gcn_b[i].reshape(1, -1), g, bb, gcn_w[i + 1])

  parts = _sc_agg(hs, sd, zeros)
  return _tc(_postdec_body, _sds(1), parts, hs, dinv,
             gcn_b[_DEPTH - 1].reshape(1, -1), dec_w1, db1, dec_w2, db2)
